# baseline (speedup 1.0000x reference)
import jax
import jax.numpy as jnp
from jax import lax
from jax.experimental import pallas as pl
from jax.experimental.pallas import tpu as pltpu
from jax.experimental.pallas import tpu_sc as plsc

NC = 2
NS = 16
NW = NC * NS
D = 128
K = 16
ROWS_PER_CHUNK = 8
IDX_PER_CHUNK = ROWS_PER_CHUNK * K
VPR = D // 16


NBUF = 2
CPS = 2


def _body(x_hbm, idx_hbm, out_hbm, idx_v, gat_v, out_v, sem0, sem1,
          osem0, osem1):
    wid = lax.axis_index("s") * NC + lax.axis_index("c")
    n_chunks = idx_hbm.shape[1]
    base_row = wid * (n_chunks * ROWS_PER_CHUNK)

    pltpu.sync_copy(idx_hbm.at[wid], idx_v)

    sems = (sem0, sem1)
    osems = (osem0, osem1)

    def start_gather(s, b, sem):
        for h in range(CPS):
            pltpu.async_copy(
                x_hbm.at[idx_v.at[CPS * s + h]],
                gat_v.at[b, pl.ds(h * IDX_PER_CHUNK, IDX_PER_CHUNK)], sem)

    def wait_gather(b, sem):
        pltpu.make_async_copy(x_hbm.at[pl.ds(0, CPS * IDX_PER_CHUNK)],
                              gat_v.at[b], sem).wait()

    start_gather(0, 0, sems[0])
    start_gather(1, 1, sems[1])

    ROWS_PER_SUPER = CPS * ROWS_PER_CHUNK
    n_super = n_chunks // CPS

    def compute_super(b):
        def row_step(r, _):
            base = r * K
            for v in range(VPR):
                col = pl.ds(v * 16, 16)
                acc = gat_v[b, base, col]
                for j in range(1, K):
                    acc = jnp.maximum(acc, gat_v[b, base + j, col])
                out_v[b, r, col] = acc
            return 0

        lax.fori_loop(0, ROWS_PER_SUPER, row_step, 0, unroll=False)

    n_out = out_hbm.shape[0]

    def wait_store(b):
        pltpu.make_async_copy(out_hbm.at[pl.ds(0, ROWS_PER_SUPER)],
                              out_v.at[b], osems[b]).wait()

    def step(g, _):
        for b in range(NBUF):
            s = g * NBUF + b
            sem = sems[b]
            wait_gather(b, sem)

            @pl.when(s >= NBUF)
            def _():
                wait_store(b)

            compute_super(b)
            for h in range(CPS):
                row0 = jnp.minimum(
                    base_row + s * ROWS_PER_SUPER + h * ROWS_PER_CHUNK,
                    n_out - ROWS_PER_CHUNK)
                pltpu.async_copy(
                    out_v.at[b, pl.ds(h * ROWS_PER_CHUNK, ROWS_PER_CHUNK)],
                    out_hbm.at[pl.ds(row0, ROWS_PER_CHUNK)], osems[b])
            next_s = s + NBUF

            @pl.when(next_s < n_super)
            def _():
                start_gather(next_s, b, sem)

        return 0

    lax.fori_loop(0, n_super // NBUF, step, 0, unroll=False)
    for b in range(NBUF):
        wait_store(b)


def kernel(x, pools):
    n2 = pools.shape[0]
    idx = pools.astype(jnp.int32)

    block = NW * ROWS_PER_CHUNK
    n_pad = ((n2 + block - 1) // block) * block
    rows_per_worker = n_pad // NW
    n_chunks = rows_per_worker // ROWS_PER_CHUNK
    while n_chunks % (CPS * NBUF) != 0:
        n_pad += block
        rows_per_worker = n_pad // NW
        n_chunks = rows_per_worker // ROWS_PER_CHUNK
    if n_pad != n2:
        tail = jnp.tile(idx[n2 - ROWS_PER_CHUNK:],
                        ((n_pad - n2) // ROWS_PER_CHUNK, 1))
        idx = jnp.concatenate([idx, tail], axis=0)

    idx_r = idx.reshape(NW, n_chunks, IDX_PER_CHUNK)

    mesh = plsc.VectorSubcoreMesh(core_axis_name="c", subcore_axis_name="s")
    run = pl.kernel(
        _body,
        out_type=jax.ShapeDtypeStruct((n2, D), jnp.float32),
        mesh=mesh,
        scratch_types=[
            pltpu.VMEM((n_chunks, IDX_PER_CHUNK), jnp.int32),
            pltpu.VMEM((NBUF, CPS * IDX_PER_CHUNK, D), jnp.float32),
            pltpu.VMEM((NBUF, CPS * ROWS_PER_CHUNK, D), jnp.float32),
            pltpu.SemaphoreType.DMA,
            pltpu.SemaphoreType.DMA,
            pltpu.SemaphoreType.DMA,
            pltpu.SemaphoreType.DMA,
        ],
    )
    return run(x, idx_r)


# speedup vs baseline: 1.0107x; 1.0107x over previous
import jax
import jax.numpy as jnp
from jax import lax
from jax.experimental import pallas as pl
from jax.experimental.pallas import tpu as pltpu
from jax.experimental.pallas import tpu_sc as plsc

NC = 2
NS = 16
NW = NC * NS
D = 128
K = 16
ROWS_PER_CHUNK = 8
IDX_PER_CHUNK = ROWS_PER_CHUNK * K
VPR = D // 16


NBUF = 2
CPS = 3


def _body(x_hbm, idx_hbm, out_hbm, idx_v, gat_v, out_v, sem0, sem1,
          osem0, osem1):
    wid = lax.axis_index("s") * NC + lax.axis_index("c")
    n_chunks = idx_hbm.shape[1]
    base_row = wid * (n_chunks * ROWS_PER_CHUNK)

    pltpu.sync_copy(idx_hbm.at[wid], idx_v)

    sems = (sem0, sem1)
    osems = (osem0, osem1)

    def start_gather(s, b, sem):
        for h in range(CPS):
            pltpu.async_copy(
                x_hbm.at[idx_v.at[CPS * s + h]],
                gat_v.at[b, pl.ds(h * IDX_PER_CHUNK, IDX_PER_CHUNK)], sem)

    def wait_gather(b, sem):
        pltpu.make_async_copy(x_hbm.at[pl.ds(0, CPS * IDX_PER_CHUNK)],
                              gat_v.at[b], sem).wait()

    start_gather(0, 0, sems[0])
    start_gather(1, 1, sems[1])

    ROWS_PER_SUPER = CPS * ROWS_PER_CHUNK
    n_super = n_chunks // CPS

    def compute_super(b):
        def row_step(r, _):
            base = r * K
            for v in range(VPR):
                col = pl.ds(v * 16, 16)
                acc = gat_v[b, base, col]
                for j in range(1, K):
                    acc = jnp.maximum(acc, gat_v[b, base + j, col])
                out_v[b, r, col] = acc
            return 0

        lax.fori_loop(0, ROWS_PER_SUPER, row_step, 0, unroll=False)

    n_out = out_hbm.shape[0]

    def wait_store(b):
        pltpu.make_async_copy(out_hbm.at[pl.ds(0, ROWS_PER_SUPER)],
                              out_v.at[b], osems[b]).wait()

    def step(g, _):
        for b in range(NBUF):
            s = g * NBUF + b
            sem = sems[b]
            wait_gather(b, sem)

            @pl.when(s >= NBUF)
            def _():
                wait_store(b)

            compute_super(b)
            for h in range(CPS):
                row0 = jnp.minimum(
                    base_row + s * ROWS_PER_SUPER + h * ROWS_PER_CHUNK,
                    n_out - ROWS_PER_CHUNK)
                pltpu.async_copy(
                    out_v.at[b, pl.ds(h * ROWS_PER_CHUNK, ROWS_PER_CHUNK)],
                    out_hbm.at[pl.ds(row0, ROWS_PER_CHUNK)], osems[b])
            next_s = s + NBUF

            @pl.when(next_s < n_super)
            def _():
                start_gather(next_s, b, sem)

        return 0

    lax.fori_loop(0, n_super // NBUF, step, 0, unroll=False)
    for b in range(NBUF):
        wait_store(b)


def kernel(x, pools):
    n2 = pools.shape[0]
    idx = pools.astype(jnp.int32)

    block = NW * ROWS_PER_CHUNK
    n_pad = ((n2 + block - 1) // block) * block
    rows_per_worker = n_pad // NW
    n_chunks = rows_per_worker // ROWS_PER_CHUNK
    while n_chunks % (CPS * NBUF) != 0:
        n_pad += block
        rows_per_worker = n_pad // NW
        n_chunks = rows_per_worker // ROWS_PER_CHUNK
    if n_pad != n2:
        tail = jnp.tile(idx[n2 - ROWS_PER_CHUNK:],
                        ((n_pad - n2) // ROWS_PER_CHUNK, 1))
        idx = jnp.concatenate([idx, tail], axis=0)

    idx_r = idx.reshape(NW, n_chunks, IDX_PER_CHUNK)

    mesh = plsc.VectorSubcoreMesh(core_axis_name="c", subcore_axis_name="s")
    run = pl.kernel(
        _body,
        out_type=jax.ShapeDtypeStruct((n2, D), jnp.float32),
        mesh=mesh,
        scratch_types=[
            pltpu.VMEM((n_chunks, IDX_PER_CHUNK), jnp.int32),
            pltpu.VMEM((NBUF, CPS * IDX_PER_CHUNK, D), jnp.float32),
            pltpu.VMEM((NBUF, CPS * ROWS_PER_CHUNK, D), jnp.float32),
            pltpu.SemaphoreType.DMA,
            pltpu.SemaphoreType.DMA,
            pltpu.SemaphoreType.DMA,
            pltpu.SemaphoreType.DMA,
        ],
    )
    return run(x, idx_r)
